# baseline (speedup 1.0000x reference)
import functools

import jax
import jax.numpy as jnp
from jax import lax
from jax.experimental import pallas as pl
from jax.experimental.pallas import tpu as pltpu
from jax.experimental.pallas import tpu_sc as plsc

EMB = 64
NC, NS = 2, 16
NW = NC * NS
TB = 128
NJ = 200
NBUF = 8
NOB = 2
LANES = 16


@functools.lru_cache(maxsize=None)
def _build_gather():
    mesh = plsc.VectorSubcoreMesh(core_axis_name="c", subcore_axis_name="s")

    def body(idx_hbm, table_hbm, out_hbm, idxblk, hidx, *scratch):
        gbufs = scratch[:NBUF]
        obufs = scratch[NBUF:NBUF + NOB]
        gsems = scratch[NBUF + NOB:2 * NBUF + NOB]
        wsems = scratch[2 * NBUF + NOB:]
        wid = lax.axis_index("s") * NC + lax.axis_index("c")

        pltpu.sync_copy(idx_hbm.at[wid], idxblk)

        iota = lax.iota(jnp.int32, LANES)
        cvecs = [iota + LANES * cb for cb in range(TB // LANES)]

        def build_hidx(j, b):
            jv = jnp.full((LANES,), j, dtype=jnp.int32)
            for cb in range(TB // LANES):
                v = plsc.load_gather(idxblk, [cvecs[cb], jv])
                hidx[b, pl.ds(LANES * cb, LANES)] = v

        def start_gather(b):
            pltpu.async_copy(table_hbm.at[hidx.at[b]], gbufs[b], gsems[b])

        def wait_gather(b):
            pltpu.make_async_copy(
                table_hbm.at[hidx.at[b]], gbufs[b], gsems[b]).wait()

        Rv = [(iota + u0) // 8 for u0 in range(0, EMB, LANES)]
        rv = [(iota + u0) % 8 for u0 in range(0, EMB, LANES)]

        def transpose(b, ob):
            gbuf, obuf = gbufs[b], obufs[ob]

            @plsc.parallel_loop(0, EMB, unroll=8)
            def _(d):
                dv = jnp.full((LANES,), d, dtype=jnp.int32)
                vs = [plsc.load_gather(gbuf, [cvecs[cb], dv])
                      for cb in range(TB // LANES)]
                for cb in range(TB // LANES):
                    obuf[d // 8, d % 8, pl.ds(LANES * cb, LANES)] = vs[cb]

        def start_write(j, ob):
            pltpu.async_copy(out_hbm.at[j, :, wid], obufs[ob], wsems[ob])

        def wait_write(j, ob):
            pltpu.make_async_copy(
                out_hbm.at[j, :, wid], obufs[ob], wsems[ob]).wait()

        for j in range(NBUF):
            build_hidx(j, j)
            start_gather(j)

        @pl.loop(0, NJ, step=NBUF)
        def _(j0):
            for k in range(NBUF):
                j = j0 + k
                ob = k % NOB
                wait_gather(k)

                @pl.when(j >= NOB)
                def _():
                    wait_write(j - NOB, ob)

                transpose(k, ob)
                start_write(j, ob)

                @pl.when(j + NBUF < NJ)
                def _():
                    build_hidx(j + NBUF, k)
                    start_gather(k)

        for k in range(NOB):
            wait_write(NJ - NOB + k, (NJ - NOB + k) % NOB)

    return pl.kernel(
        body,
        mesh=mesh,
        compiler_params=pltpu.CompilerParams(
            use_tc_tiling_on_sc=False, needs_layout_passes=False),
        out_type=jax.ShapeDtypeStruct((NJ, EMB // 8, NW, 8, TB), jnp.float32),
        scratch_types=(
            [pltpu.VMEM((TB, NJ), jnp.int32),
             pltpu.VMEM((NBUF, TB), jnp.int32)]
            + [pltpu.VMEM((TB, EMB), jnp.float32)] * NBUF
            + [pltpu.VMEM((EMB // 8, 8, TB), jnp.float32)] * NOB
            + [pltpu.SemaphoreType.DMA] * (NBUF + NOB)
        ),
    )


def kernel(token_ids, weight):
    ni, nj = token_ids.shape
    flat = token_ids.reshape(-1).astype(jnp.int32)
    idx3 = flat.reshape(NW, TB, NJ)
    x = _build_gather()(idx3, weight)
    out = x.transpose(2, 4, 0, 1, 3).reshape(ni, nj, EMB)
    return out


# speedup vs baseline: 1.2483x; 1.2483x over previous
import functools

import jax
import jax.numpy as jnp
from jax import lax
from jax.experimental import pallas as pl
from jax.experimental.pallas import tpu as pltpu
from jax.experimental.pallas import tpu_sc as plsc

EMB_DIM = 64
NC, NS = 2, 16
NW = NC * NS
CHUNK = 128
NBUF = 8
K = 4


@functools.lru_cache(maxsize=None)
def _build_gather(n_chunks: int):
    b_per_w = n_chunks * CHUNK
    n_rows = NW * b_per_w
    mesh = plsc.VectorSubcoreMesh(core_axis_name="c", subcore_axis_name="s")

    def body(idx_hbm, table_hbm, out_hbm, idx_v, *scratch):
        rows = scratch[:NBUF]
        gsems = scratch[NBUF:2 * NBUF]
        wsems = scratch[2 * NBUF:3 * NBUF]
        wid = lax.axis_index("s") * NC + lax.axis_index("c")
        base = wid * b_per_w

        pltpu.sync_copy(idx_hbm.at[wid], idx_v)

        def wait_gather(c, b):
            pltpu.make_async_copy(
                table_hbm.at[idx_v.at[c]], rows[b], gsems[b]).wait()

        def start_write(c, b):
            pltpu.async_copy(
                rows[b], out_hbm.at[pl.ds(base + c * CHUNK, CHUNK)],
                wsems[b])

        def wait_write(c, b):
            pltpu.make_async_copy(
                rows[b], out_hbm.at[pl.ds(base + c * CHUNK, CHUNK)],
                wsems[b]).wait()

        def start_gather(c, b):
            pltpu.async_copy(table_hbm.at[idx_v.at[c]], rows[b], gsems[b])

        for b in range(K):
            start_gather(b, b)

        for c in range(K):
            wait_gather(c, c)
            start_write(c, c)
            start_gather(c + K, c + K)

        @pl.loop(K, n_chunks - K, step=NBUF)
        def _(c0):
            for j in range(NBUF):
                c = c0 + j
                b = (K + j) % NBUF
                pb = (2 * K + j) % NBUF
                wait_gather(c, b)
                start_write(c, b)
                wait_write(c - K, pb)
                start_gather(c + K, pb)

        for cs in range(n_chunks - K, n_chunks):
            b = cs % NBUF
            wait_gather(cs, b)
            start_write(cs, b)

        for cs in range(n_chunks - NBUF, n_chunks):
            wait_write(cs, cs % NBUF)

    return pl.kernel(
        body,
        mesh=mesh,
        compiler_params=pltpu.CompilerParams(use_tc_tiling_on_sc=False),
        out_type=jax.ShapeDtypeStruct((n_rows, EMB_DIM), jnp.float32),
        scratch_types=(
            [pltpu.VMEM((n_chunks, CHUNK), jnp.int32)]
            + [pltpu.VMEM((CHUNK, EMB_DIM), jnp.float32)] * NBUF
            + [pltpu.SemaphoreType.DMA] * (2 * NBUF)
        ),
    )


def kernel(token_ids, weight):
    orig_shape = token_ids.shape
    flat = token_ids.reshape(-1).astype(jnp.int32)
    n = flat.shape[0]
    tile = NW * CHUNK * NBUF
    n_pad = -(-n // tile) * tile
    if n_pad != n:
        flat = jnp.pad(flat, (0, n_pad - n))
    n_chunks = n_pad // (NW * CHUNK)
    idx3 = flat.reshape(NW, n_chunks, CHUNK)
    out = _build_gather(n_chunks)(idx3, weight)
    if n_pad != n:
        out = out[:n]
    return out.reshape(*orig_shape, EMB_DIM)
